# baseline (speedup 1.0000x reference)
import jax
import jax.numpy as jnp
from jax import lax
from jax.experimental import pallas as pl
from jax.experimental.pallas import tpu as pltpu
from jax.experimental.pallas import tpu_sc as plsc

NUM_FEATURES = 32
BATCH = 16384

NC = 2
NS = 16
NW = NC * NS
LANES = 16

B_PER_W = BATCH // NW
IDX_CHUNK = 128
N_CHUNKS = B_PER_W // IDX_CHUNK


def _mf_body(u_hbm, v_hbm, ue_hbm, ie_hbm, out_hbm,
             uidx_v, vidx_v, urows_v, vrows_v, out_v, sem):
    wid = lax.axis_index("s") * NC + lax.axis_index("c")
    base = wid * B_PER_W

    for j in range(N_CHUNKS):
        pltpu.sync_copy(u_hbm.at[pl.ds(base + j * IDX_CHUNK, IDX_CHUNK)],
                        uidx_v.at[j])
        pltpu.sync_copy(v_hbm.at[pl.ds(base + j * IDX_CHUNK, IDX_CHUNK)],
                        vidx_v.at[j])

    copies = []
    for j in range(N_CHUNKS):
        dst = urows_v.at[pl.ds(j * IDX_CHUNK, IDX_CHUNK)]
        copies.append(pltpu.async_copy(ue_hbm.at[uidx_v.at[j]], dst, sem))
        dst = vrows_v.at[pl.ds(j * IDX_CHUNK, IDX_CHUNK)]
        copies.append(pltpu.async_copy(ie_hbm.at[vidx_v.at[j]], dst, sem))
    for c in copies:
        c.wait()

    def chunk(i, carry):
        rows = lax.iota(jnp.int32, LANES) + i * LANES
        acc = jnp.zeros((LANES,), jnp.float32)
        for f in range(NUM_FEATURES):
            col = jnp.full((LANES,), f, jnp.int32)
            a = plsc.load_gather(urows_v, [rows, col])
            b = plsc.load_gather(vrows_v, [rows, col])
            acc = acc + a * b
        out_v[pl.ds(i * LANES, LANES)] = acc
        return carry

    lax.fori_loop(0, B_PER_W // LANES, chunk, 0)

    pltpu.sync_copy(out_v, out_hbm.at[pl.ds(base, B_PER_W)])


@jax.jit
def kernel(u, v, user_embed, item_embed):
    mesh = plsc.VectorSubcoreMesh(core_axis_name="c", subcore_axis_name="s")
    f = pl.kernel(
        _mf_body,
        out_type=jax.ShapeDtypeStruct((BATCH,), jnp.float32),
        mesh=mesh,
        scratch_types=[
            pltpu.VMEM((N_CHUNKS, IDX_CHUNK), jnp.int32),
            pltpu.VMEM((N_CHUNKS, IDX_CHUNK), jnp.int32),
            pltpu.VMEM((B_PER_W, NUM_FEATURES), jnp.float32),
            pltpu.VMEM((B_PER_W, NUM_FEATURES), jnp.float32),
            pltpu.VMEM((B_PER_W,), jnp.float32),
            pltpu.SemaphoreType.DMA,
        ],
        compiler_params=pltpu.CompilerParams(
            needs_layout_passes=False, use_tc_tiling_on_sc=False),
    )
    return f(u, v, user_embed, item_embed)
